# baseline (speedup 1.0000x reference)
import functools
import jax
import jax.numpy as jnp
from jax import lax
from jax.experimental import pallas as pl
from jax.experimental.pallas import tpu as pltpu
import jax.experimental.pallas.tpu_sc as plsc

_N, _E, _D = 10000, 320000, 128
_NC, _NS, _L = 2, 16, 16
_NW = _NC * _NS
_EP = 327680
_EPT = _EP // _NW
_CH = 2048
_KB = 128
_NBLK = _CH // _KB
_ER = _EP // _KB
_NP = 10240
_NPS = _NP // _NS

_SC_PARAMS = pltpu.CompilerParams(needs_layout_passes=False,
                                  use_tc_tiling_on_sc=False)


def _edge_scalar_body(src2_h, dst2_h, el_h, es_h, ed_h, emax_h, z2_h,
                      ew_h, g_h, dd_h,
                      es_t, ed_t, emax_t, src_c2, dst_c2, el_c, ew_c, g_c,
                      vals2, acc, sem_s):
    c = lax.axis_index("c")
    s = lax.axis_index("s")
    wid = s * _NC + c
    pltpu.sync_copy(es_h, es_t)
    pltpu.sync_copy(ed_h, ed_t)
    pltpu.sync_copy(emax_h, emax_t)
    nrow0 = s * _NPS
    pltpu.sync_copy(z2_h.at[pl.ds(nrow0, _NPS), :],
                    acc.at[pl.ds(nrow0, _NPS), :])
    plsc.subcore_barrier()
    iota = lax.iota(jnp.int32, 16)
    col0 = jnp.zeros((16,), jnp.int32)
    col1 = jnp.ones((16,), jnp.int32)
    emax_v = emax_t[...]
    ebase = wid * _EPT
    rbase = wid * (_EPT // _KB)

    def chunk(ch, _):
        base = ebase + ch * _CH
        rb = rbase + ch * _NBLK
        pltpu.sync_copy(src2_h.at[pl.ds(rb, _NBLK), :], src_c2)
        pltpu.sync_copy(dst2_h.at[pl.ds(rb, _NBLK), :], dst_c2)
        pltpu.sync_copy(el_h.at[pl.ds(base, _CH)], el_c)

        def grp(k, _):
            o = k * 16
            row = jnp.full((16,), k >> 3, jnp.int32)
            colb = (k & 7) * 16 + iota
            src16 = plsc.load_gather(src_c2, [row, colb])
            dst16 = plsc.load_gather(dst_c2, [row, colb])
            el16 = el_c[pl.ds(o, 16)]
            esg = plsc.load_gather(es_t, [src16])
            edg = plsc.load_gather(ed_t, [dst16])
            t = esg + edg
            e = jnp.where(t >= 0, t, t * jnp.float32(0.2))
            ewv = 1.0 / (1.0 + jnp.exp(-el16))
            gv = jnp.exp(e - emax_v) * ewv
            valid = (base + o + iota) < _E
            ewv = jnp.where(valid, ewv, 0.0)
            gv = jnp.where(valid, gv, 0.0)
            ew_c[pl.ds(o, 16)] = ewv
            g_c[pl.ds(o, 16)] = gv
            ridx = o + iota
            plsc.store_scatter(vals2, [ridx, col0], ewv)
            plsc.store_scatter(vals2, [ridx, col1], gv)
            return 0

        lax.fori_loop(0, _CH // 16, grp, 0)
        pltpu.sync_copy(ew_c, ew_h.at[pl.ds(base, _CH)])
        pltpu.sync_copy(g_c, g_h.at[pl.ds(base, _CH)])

        def fire(b, _):
            pltpu.async_copy(vals2.at[pl.ds(b * _KB, _KB), :],
                             acc.at[dst_c2.at[b]], sem_s, add=True)
            return 0

        lax.fori_loop(0, _NBLK, fire, 0)

        def drain(b, _):
            pltpu.make_async_copy(vals2.at[pl.ds(b * _KB, _KB), :],
                                  acc.at[dst_c2.at[b]], sem_s).wait()
            return 0

        lax.fori_loop(0, _NBLK, drain, 0)
        return 0

    lax.fori_loop(0, _EPT // _CH, chunk, 0)
    plsc.subcore_barrier()
    pltpu.sync_copy(acc.at[pl.ds(nrow0, _NPS), :],
                    dd_h.at[c, pl.ds(nrow0, _NPS), :])


def _edge_scalar_pass(src2, dst2, elp, es, ed, emax):
    emax16 = jnp.full((16,), emax, jnp.float32)
    zeros2 = jnp.zeros((_NP, 2), jnp.float32)
    f32 = jnp.float32
    kern = pl.kernel(
        _edge_scalar_body,
        out_type=(jax.ShapeDtypeStruct((_EP,), f32),
                  jax.ShapeDtypeStruct((_EP,), f32),
                  jax.ShapeDtypeStruct((_NC, _NP, 2), f32)),
        mesh=plsc.VectorSubcoreMesh(core_axis_name="c", subcore_axis_name="s"),
        compiler_params=_SC_PARAMS,
        scratch_types=[
            pltpu.VMEM((_N,), f32),
            pltpu.VMEM((_N,), f32),
            pltpu.VMEM((16,), f32),
            pltpu.VMEM((_NBLK, _KB), jnp.int32),
            pltpu.VMEM((_NBLK, _KB), jnp.int32),
            pltpu.VMEM((_CH,), f32),
            pltpu.VMEM((_CH,), f32),
            pltpu.VMEM((_CH,), f32),
            pltpu.VMEM((_CH, 2), f32),
            pltpu.VMEM_SHARED((_NP, 2), f32),
            pltpu.SemaphoreType.DMA,
        ],
    )
    return kern(src2, dst2, elp, es, ed, emax16, zeros2)


_KD = 80
_CHD = 2048


def _agg_body(with_dinv, *refs):
    if with_dinv:
        (src_h, dst_h, cf_h, dinv_h, x_h, out_h,
         dinv_t, src_c, dst_c, cf_c, src_blk, dst_blk, rows, sc, acc) = refs
    else:
        (src_h, dst_h, cf_h, x_h, out_h,
         src_c, dst_c, cf_c, src_blk, dst_blk, rows, sc, acc) = refs
        dinv_t = None
    c = lax.axis_index("c")
    s = lax.axis_index("s")
    wid = s * _NC + c
    nrow0 = s * _NPS
    if with_dinv:
        pltpu.sync_copy(dinv_h, dinv_t)
    iota = lax.iota(jnp.int32, 16)
    zero16 = jnp.zeros((16,), jnp.float32)

    def zrow(e, _):
        re = jnp.full((16,), e, jnp.int32)
        for j in range(8):
            plsc.store_scatter(sc, [re, j * 16 + iota], zero16)
        return 0

    lax.fori_loop(0, _KD, zrow, 0)
    for q in range(_NPS // _KD):
        pltpu.sync_copy(sc, acc.at[pl.ds(nrow0 + q * _KD, _KD), :])
    plsc.subcore_barrier()
    ebase = wid * _EPT

    def chunk(ch, _):
        base = ebase + ch * _CHD
        pltpu.sync_copy(src_h.at[pl.ds(base, _CHD)], src_c)
        pltpu.sync_copy(dst_h.at[pl.ds(base, _CHD)], dst_c)
        pltpu.sync_copy(cf_h.at[pl.ds(base, _CHD)], cf_c)

        def block(b, _):
            o = b * _KD
            for j in range(_KD // 16):
                src_blk[pl.ds(j * 16, 16)] = src_c[pl.ds(o + j * 16, 16)]
                dst_blk[pl.ds(j * 16, 16)] = dst_c[pl.ds(o + j * 16, 16)]
            pltpu.sync_copy(x_h.at[src_blk], rows)

            def edge(e, _):
                ei = jnp.full((16,), o + e, jnp.int32)
                csp = plsc.load_gather(cf_c, [ei])
                if with_dinv:
                    srcv = plsc.load_gather(src_c, [ei])
                    csp = csp * plsc.load_gather(dinv_t, [srcv])
                re = jnp.full((16,), e, jnp.int32)
                for j in range(8):
                    colj = j * 16 + iota
                    r = plsc.load_gather(rows, [re, colj])
                    plsc.store_scatter(sc, [re, colj], r * csp)
                return 0

            lax.fori_loop(0, _KD, edge, 0)
            pltpu.sync_copy(sc, acc.at[dst_blk], add=True)
            return 0

        lax.fori_loop(0, _CHD // _KD, block, 0)
        return 0

    lax.fori_loop(0, _EPT // _CHD, chunk, 0)
    plsc.subcore_barrier()
    pltpu.sync_copy(acc.at[pl.ds(nrow0, _NPS), :],
                    out_h.at[c, pl.ds(nrow0, _NPS), :])


def _feature_agg_pass(src, dst, coef, x, dinv=None):
    f32 = jnp.float32
    with_dinv = dinv is not None
    scratch = [
        pltpu.VMEM((_CHD,), jnp.int32),
        pltpu.VMEM((_CHD,), jnp.int32),
        pltpu.VMEM((_CHD,), f32),
        pltpu.VMEM((_KD,), jnp.int32),
        pltpu.VMEM((_KD,), jnp.int32),
        pltpu.VMEM((_KD, _D), f32),
        pltpu.VMEM((_KD, _D), f32),
        pltpu.VMEM_SHARED((_NP, _D), f32),
    ]
    if with_dinv:
        scratch = [pltpu.VMEM((_N,), f32)] + scratch
    kern = pl.kernel(
        functools.partial(_agg_body, with_dinv),
        out_type=jax.ShapeDtypeStruct((_NC, _NP, _D), f32),
        mesh=plsc.VectorSubcoreMesh(core_axis_name="c", subcore_axis_name="s"),
        compiler_params=_SC_PARAMS,
        scratch_types=scratch,
    )
    if with_dinv:
        return kern(src, dst, coef, dinv, x)
    return kern(src, dst, coef, x)


def _esed_body(W_ref, ad_ref, x_ref, o_ref):
    v = jnp.dot(W_ref[...], ad_ref[...].T, preferred_element_type=jnp.float32)
    o_ref[...] = jnp.dot(x_ref[...], v, preferred_element_type=jnp.float32)


def _esed(x, W, a_src, a_dst, bn=2000):
    n, d = x.shape
    ad = jnp.stack([a_src, a_dst], axis=0)
    return pl.pallas_call(
        _esed_body,
        grid=(n // bn,),
        in_specs=[
            pl.BlockSpec((d, d), lambda i: (0, 0)),
            pl.BlockSpec((2, d), lambda i: (0, 0)),
            pl.BlockSpec((bn, d), lambda i: (i, 0)),
        ],
        out_specs=pl.BlockSpec((bn, 2), lambda i: (i, 0)),
        out_shape=jax.ShapeDtypeStruct((n, 2), jnp.float32),
    )(W, ad, x)


def _combine_body(relu_in, p_ref, x_ref, aa0_ref, aa1_ref, an0_ref, an1_ref,
                  aw0_ref, aw1_ref, nv_ref,
                  Wgat_ref, Wgcn_ref, Wgin_ref, Wss_ref, Wsn_ref, Wlin_ref,
                  b_ref, o_ref):
    w0, w1, w2, w3, w4, onep_eps = (p_ref[0], p_ref[1], p_ref[2], p_ref[3],
                                    p_ref[4], p_ref[5])
    x = x_ref[...]
    if relu_in:
        x = jnp.maximum(x, 0.0)
    denom = nv_ref[:, 0:1]
    dinv = nv_ref[:, 1:2]
    cnt = nv_ref[:, 2:3]
    aa = aa0_ref[...] + aa1_ref[...]
    an = an0_ref[...] + an1_ref[...]
    aw = aw0_ref[...] + aw1_ref[...]
    gat_in = aa / (denom + 1e-16)
    gcn_in = an * dinv + x * (dinv * dinv)
    gin_in = onep_eps * x + aw
    mean = aw / (cnt + 1e-16)

    f32 = jnp.float32
    acc = w0 * jnp.dot(gat_in, Wgat_ref[...], preferred_element_type=f32)
    acc += w1 * jnp.dot(gcn_in, Wgcn_ref[...], preferred_element_type=f32)
    acc += w2 * jnp.dot(gin_in, Wgin_ref[...], preferred_element_type=f32)
    acc += w3 * (jnp.dot(x, Wss_ref[...], preferred_element_type=f32)
                 + jnp.dot(mean, Wsn_ref[...], preferred_element_type=f32))
    acc += w4 * jnp.dot(x, Wlin_ref[...], preferred_element_type=f32)
    bias = (w0 * b_ref[0:1, :] + w1 * b_ref[1:2, :] + w2 * b_ref[2:3, :]
            + w3 * b_ref[3:4, :] + w4 * b_ref[4:5, :])
    o_ref[...] = acc + bias


def _combine(x, agg, nodevec, params, Ws, biases, relu_in, bn=2000):
    n, d = x.shape
    wspec = pl.BlockSpec((d, d), lambda i: (0, 0))
    hspec = pl.BlockSpec((bn, d), lambda i: (i, 0))
    agg_a, agg_n, agg_w = agg
    halves = (agg_a[0], agg_a[1], agg_n[0], agg_n[1], agg_w[0], agg_w[1])
    return pl.pallas_call(
        functools.partial(_combine_body, relu_in),
        grid=(n // bn,),
        in_specs=[
            pl.BlockSpec(memory_space=pltpu.SMEM),
            pl.BlockSpec((bn, d), lambda i: (i, 0)),
            hspec, hspec, hspec, hspec, hspec, hspec,
            pl.BlockSpec((bn, 4), lambda i: (i, 0)),
            wspec, wspec, wspec, wspec, wspec, wspec,
            pl.BlockSpec((5, d), lambda i: (0, 0)),
        ],
        out_specs=pl.BlockSpec((bn, d), lambda i: (i, 0)),
        out_shape=jax.ShapeDtypeStruct((n, d), jnp.float32),
    )(params, x, *halves, nodevec, *Ws, biases)


def _step(x, src, dst, el, w, eps, Ws, biases, gat_W, a_src, a_dst, relu_in):
    n, d = x.shape
    xr = jnp.maximum(x, 0.0) if relu_in else x

    pad = _EP - _E
    src2 = jnp.concatenate([src, jnp.zeros((pad,), src.dtype)]).reshape(
        _ER, _KB)
    dst2 = jnp.concatenate([dst, jnp.zeros((pad,), dst.dtype)]).reshape(
        _ER, _KB)
    elp = jnp.concatenate([el, jnp.zeros((pad,), el.dtype)])

    esed = _esed(xr, gat_W, a_src, a_dst)
    es, ed = esed[:, 0], esed[:, 1]
    emax = jax.nn.leaky_relu(jnp.max(es) + jnp.max(ed), 0.2)

    ew, g, dd = _edge_scalar_pass(src2, dst2, elp, es, ed, emax)
    deg = dd[0, :_N, 0] + dd[1, :_N, 0] + 1.0
    denom = dd[0, :_N, 1] + dd[1, :_N, 1]
    dinv = jax.lax.rsqrt(deg + 1e-16)

    srcp = src2.reshape(_EP)
    dstp = dst2.reshape(_EP)
    agg_a = _feature_agg_pass(srcp, dstp, g, xr)
    agg_n = _feature_agg_pass(srcp, dstp, ew, xr, dinv=dinv)
    agg_w = _feature_agg_pass(srcp, dstp, ew, xr)
    agg = (agg_a, agg_n, agg_w)

    cnt = deg - 1.0
    nodevec = jnp.stack([denom, dinv, cnt, cnt], axis=1)
    params = jnp.concatenate([w, jnp.reshape(1.0 + eps, (1,))])
    return _combine(x, agg, nodevec, params, Ws, biases, relu_in)


def kernel(x, edge_index0, edge_logits0, edge_index1, edge_logits1, weights,
           gcn_W, gcn_b, gat_W, gat_a_src, gat_a_dst, gat_b,
           gin_W, gin_b, gin_eps, sage_Ws, sage_Wn, sage_b, lin_W, lin_b):
    h = x
    for i, (ei, el) in enumerate(((edge_index0, edge_logits0),
                                  (edge_index1, edge_logits1))):
        Ws = (gat_W[i], gcn_W[i], gin_W[i], sage_Ws[i], sage_Wn[i], lin_W[i])
        biases = jnp.stack([gat_b[i], gcn_b[i], gin_b[i], sage_b[i],
                            lin_b[i]], axis=0)
        h = _step(h, ei[0], ei[1], el, weights[i], gin_eps[i], Ws, biases,
                  gat_W[i], gat_a_src[i], gat_a_dst[i], relu_in=(i == 1))
    return h


# speedup vs baseline: 1.4820x; 1.4820x over previous
import functools
import jax
import jax.numpy as jnp
from jax import lax
from jax.experimental import pallas as pl
from jax.experimental.pallas import tpu as pltpu
import jax.experimental.pallas.tpu_sc as plsc

_N, _E, _D = 10000, 320000, 128
_NC, _NS, _L = 2, 16, 16
_NW = _NC * _NS
_EPT = _E // _NW
_CH = 2000
_KB = 80
_NP = 10240
_NPS = _NP // _NS

_SC_PARAMS = pltpu.CompilerParams(needs_layout_passes=False,
                                  use_tc_tiling_on_sc=False)


def _edge_scalar_body(src_h, dst_h, el_h, es_h, ed_h, emax_h, z2_h,
                      ew_h, g_h, dd_h,
                      es_t, ed_t, emax_t, src_c, dst_c, el_c, ew_c, g_c,
                      vals2, dst_blk, acc):
    c = lax.axis_index("c")
    s = lax.axis_index("s")
    wid = s * _NC + c
    pltpu.sync_copy(es_h, es_t)
    pltpu.sync_copy(ed_h, ed_t)
    pltpu.sync_copy(emax_h, emax_t)
    nrow0 = s * _NPS
    pltpu.sync_copy(z2_h.at[pl.ds(nrow0, _NPS), :],
                    acc.at[pl.ds(nrow0, _NPS), :])
    plsc.subcore_barrier()
    iota = lax.iota(jnp.int32, 16)
    col0 = jnp.zeros((16,), jnp.int32)
    col1 = jnp.ones((16,), jnp.int32)
    emax_v = emax_t[...]
    ebase = wid * _EPT
    for ch in range(_EPT // _CH):
        base = ebase + ch * _CH
        pltpu.sync_copy(src_h.at[pl.ds(base, _CH)], src_c)
        pltpu.sync_copy(dst_h.at[pl.ds(base, _CH)], dst_c)
        pltpu.sync_copy(el_h.at[pl.ds(base, _CH)], el_c)

        def grp(k, _):
            o = k * 16
            src16 = src_c[pl.ds(o, 16)]
            dst16 = dst_c[pl.ds(o, 16)]
            el16 = el_c[pl.ds(o, 16)]
            esg = plsc.load_gather(es_t, [src16])
            edg = plsc.load_gather(ed_t, [dst16])
            t = esg + edg
            e = jnp.where(t >= 0, t, t * jnp.float32(0.2))
            ewv = 1.0 / (1.0 + jnp.exp(-el16))
            gv = jnp.exp(e - emax_v) * ewv
            ew_c[pl.ds(o, 16)] = ewv
            g_c[pl.ds(o, 16)] = gv
            ridx = o + iota
            plsc.store_scatter(vals2, [ridx, col0], ewv)
            plsc.store_scatter(vals2, [ridx, col1], gv)
            return 0

        lax.fori_loop(0, _CH // 16, grp, 0)
        pltpu.sync_copy(ew_c, ew_h.at[pl.ds(base, _CH)])
        pltpu.sync_copy(g_c, g_h.at[pl.ds(base, _CH)])
        for b in range(_CH // _KB):
            for j in range(_KB // 16):
                dst_blk[pl.ds(j * 16, 16)] = dst_c[pl.ds(b * _KB + j * 16, 16)]
            pltpu.sync_copy(vals2.at[pl.ds(b * _KB, _KB), :],
                            acc.at[dst_blk], add=True)
    plsc.subcore_barrier()
    pltpu.sync_copy(acc.at[pl.ds(nrow0, _NPS), :],
                    dd_h.at[c, pl.ds(nrow0, _NPS), :])


def _edge_scalar_pass(src, dst, el, es, ed, emax):
    emax16 = jnp.full((16,), emax, jnp.float32)
    zeros2 = jnp.zeros((_NP, 2), jnp.float32)
    f32 = jnp.float32
    kern = pl.kernel(
        _edge_scalar_body,
        out_type=(jax.ShapeDtypeStruct((_E,), f32),
                  jax.ShapeDtypeStruct((_E,), f32),
                  jax.ShapeDtypeStruct((_NC, _NP, 2), f32)),
        mesh=plsc.VectorSubcoreMesh(core_axis_name="c", subcore_axis_name="s"),
        compiler_params=_SC_PARAMS,
        scratch_types=[
            pltpu.VMEM((_N,), f32),
            pltpu.VMEM((_N,), f32),
            pltpu.VMEM((16,), f32),
            pltpu.VMEM((_CH,), jnp.int32),
            pltpu.VMEM((_CH,), jnp.int32),
            pltpu.VMEM((_CH,), f32),
            pltpu.VMEM((_CH,), f32),
            pltpu.VMEM((_CH,), f32),
            pltpu.VMEM((_CH, 2), f32),
            pltpu.VMEM((_KB,), jnp.int32),
            pltpu.VMEM_SHARED((_NP, 2), f32),
        ],
    )
    return kern(src, dst, el, es, ed, emax16, zeros2)


_KD = 80
_CHD = 2000


def _agg_body(with_dinv, *refs):
    if with_dinv:
        (src_h, dst_h, cf_h, dinv_h, x_h, out_h,
         dinv_t, src_c, dst_c, cf_c, src_blk, dst_blk, rows, sc, acc) = refs
    else:
        (src_h, dst_h, cf_h, x_h, out_h,
         src_c, dst_c, cf_c, src_blk, dst_blk, rows, sc, acc) = refs
        dinv_t = None
    c = lax.axis_index("c")
    s = lax.axis_index("s")
    wid = s * _NC + c
    nrow0 = s * _NPS
    if with_dinv:
        pltpu.sync_copy(dinv_h, dinv_t)
    iota = lax.iota(jnp.int32, 16)
    zero16 = jnp.zeros((16,), jnp.float32)

    def zrow(e, _):
        re = jnp.full((16,), e, jnp.int32)
        for j in range(8):
            plsc.store_scatter(sc, [re, j * 16 + iota], zero16)
        return 0

    lax.fori_loop(0, _KD, zrow, 0)
    for q in range(_NPS // _KD):
        pltpu.sync_copy(sc, acc.at[pl.ds(nrow0 + q * _KD, _KD), :])
    plsc.subcore_barrier()
    ebase = wid * _EPT

    def chunk(ch, _):
        base = ebase + ch * _CHD
        pltpu.sync_copy(src_h.at[pl.ds(base, _CHD)], src_c)
        pltpu.sync_copy(dst_h.at[pl.ds(base, _CHD)], dst_c)
        pltpu.sync_copy(cf_h.at[pl.ds(base, _CHD)], cf_c)

        def block(b, _):
            o = b * _KD
            for j in range(_KD // 16):
                src_blk[pl.ds(j * 16, 16)] = src_c[pl.ds(o + j * 16, 16)]
                dst_blk[pl.ds(j * 16, 16)] = dst_c[pl.ds(o + j * 16, 16)]
            pltpu.sync_copy(x_h.at[src_blk], rows)

            def edge(e, _):
                ei = jnp.full((16,), o + e, jnp.int32)
                csp = plsc.load_gather(cf_c, [ei])
                if with_dinv:
                    srcv = plsc.load_gather(src_c, [ei])
                    csp = csp * plsc.load_gather(dinv_t, [srcv])
                re = jnp.full((16,), e, jnp.int32)
                for j in range(8):
                    colj = j * 16 + iota
                    r = plsc.load_gather(rows, [re, colj])
                    plsc.store_scatter(sc, [re, colj], r * csp)
                return 0

            lax.fori_loop(0, _KD, edge, 0)
            pltpu.sync_copy(sc, acc.at[dst_blk], add=True)
            return 0

        lax.fori_loop(0, _CHD // _KD, block, 0)
        return 0

    lax.fori_loop(0, _EPT // _CHD, chunk, 0)
    plsc.subcore_barrier()
    pltpu.sync_copy(acc.at[pl.ds(nrow0, _NPS), :],
                    out_h.at[c, pl.ds(nrow0, _NPS), :])


def _feature_agg_pass(src, dst, coef, x, dinv=None):
    f32 = jnp.float32
    with_dinv = dinv is not None
    scratch = [
        pltpu.VMEM((_CHD,), jnp.int32),
        pltpu.VMEM((_CHD,), jnp.int32),
        pltpu.VMEM((_CHD,), f32),
        pltpu.VMEM((_KD,), jnp.int32),
        pltpu.VMEM((_KD,), jnp.int32),
        pltpu.VMEM((_KD, _D), f32),
        pltpu.VMEM((_KD, _D), f32),
        pltpu.VMEM_SHARED((_NP, _D), f32),
    ]
    if with_dinv:
        scratch = [pltpu.VMEM((_N,), f32)] + scratch
    kern = pl.kernel(
        functools.partial(_agg_body, with_dinv),
        out_type=jax.ShapeDtypeStruct((_NC, _NP, _D), f32),
        mesh=plsc.VectorSubcoreMesh(core_axis_name="c", subcore_axis_name="s"),
        compiler_params=_SC_PARAMS,
        scratch_types=scratch,
    )
    if with_dinv:
        return kern(src, dst, coef, dinv, x)
    return kern(src, dst, coef, x)


def _esed_body(W_ref, ad_ref, x_ref, o_ref):
    v = jnp.dot(W_ref[...], ad_ref[...].T, preferred_element_type=jnp.float32)
    o_ref[...] = jnp.dot(x_ref[...], v, preferred_element_type=jnp.float32)


def _esed(x, W, a_src, a_dst, bn=2000):
    n, d = x.shape
    ad = jnp.stack([a_src, a_dst], axis=0)
    return pl.pallas_call(
        _esed_body,
        grid=(n // bn,),
        in_specs=[
            pl.BlockSpec((d, d), lambda i: (0, 0)),
            pl.BlockSpec((2, d), lambda i: (0, 0)),
            pl.BlockSpec((bn, d), lambda i: (i, 0)),
        ],
        out_specs=pl.BlockSpec((bn, 2), lambda i: (i, 0)),
        out_shape=jax.ShapeDtypeStruct((n, 2), jnp.float32),
    )(W, ad, x)


def _combine_body(relu_in, p_ref, x_ref, aa0_ref, aa1_ref, an0_ref, an1_ref,
                  aw0_ref, aw1_ref, nv_ref,
                  Wgat_ref, Wgcn_ref, Wgin_ref, Wss_ref, Wsn_ref, Wlin_ref,
                  b_ref, o_ref):
    w0, w1, w2, w3, w4, onep_eps = (p_ref[0], p_ref[1], p_ref[2], p_ref[3],
                                    p_ref[4], p_ref[5])
    x = x_ref[...]
    if relu_in:
        x = jnp.maximum(x, 0.0)
    denom = nv_ref[:, 0:1]
    dinv = nv_ref[:, 1:2]
    cnt = nv_ref[:, 2:3]
    aa = aa0_ref[...] + aa1_ref[...]
    an = an0_ref[...] + an1_ref[...]
    aw = aw0_ref[...] + aw1_ref[...]
    gat_in = aa / (denom + 1e-16)
    gcn_in = an * dinv + x * (dinv * dinv)
    gin_in = onep_eps * x + aw
    mean = aw / (cnt + 1e-16)

    f32 = jnp.float32
    acc = w0 * jnp.dot(gat_in, Wgat_ref[...], preferred_element_type=f32)
    acc += w1 * jnp.dot(gcn_in, Wgcn_ref[...], preferred_element_type=f32)
    acc += w2 * jnp.dot(gin_in, Wgin_ref[...], preferred_element_type=f32)
    acc += w3 * (jnp.dot(x, Wss_ref[...], preferred_element_type=f32)
                 + jnp.dot(mean, Wsn_ref[...], preferred_element_type=f32))
    acc += w4 * jnp.dot(x, Wlin_ref[...], preferred_element_type=f32)
    bias = (w0 * b_ref[0:1, :] + w1 * b_ref[1:2, :] + w2 * b_ref[2:3, :]
            + w3 * b_ref[3:4, :] + w4 * b_ref[4:5, :])
    o_ref[...] = acc + bias


def _combine(x, agg, nodevec, params, Ws, biases, relu_in, bn=2000):
    n, d = x.shape
    wspec = pl.BlockSpec((d, d), lambda i: (0, 0))
    hspec = pl.BlockSpec((bn, d), lambda i: (i, 0))
    agg_a, agg_n, agg_w = agg
    halves = (agg_a[0], agg_a[1], agg_n[0], agg_n[1], agg_w[0], agg_w[1])
    return pl.pallas_call(
        functools.partial(_combine_body, relu_in),
        grid=(n // bn,),
        in_specs=[
            pl.BlockSpec(memory_space=pltpu.SMEM),
            pl.BlockSpec((bn, d), lambda i: (i, 0)),
            hspec, hspec, hspec, hspec, hspec, hspec,
            pl.BlockSpec((bn, 4), lambda i: (i, 0)),
            wspec, wspec, wspec, wspec, wspec, wspec,
            pl.BlockSpec((5, d), lambda i: (0, 0)),
        ],
        out_specs=pl.BlockSpec((bn, d), lambda i: (i, 0)),
        out_shape=jax.ShapeDtypeStruct((n, d), jnp.float32),
    )(params, x, *halves, nodevec, *Ws, biases)


def _step(x, src, dst, el, w, eps, Ws, biases, gat_W, a_src, a_dst, relu_in):
    n, d = x.shape
    xr = jnp.maximum(x, 0.0) if relu_in else x

    esed = _esed(xr, gat_W, a_src, a_dst)
    es, ed = esed[:, 0], esed[:, 1]
    emax = jax.nn.leaky_relu(jnp.max(es) + jnp.max(ed), 0.2)

    ew, g, dd = _edge_scalar_pass(src, dst, el, es, ed, emax)
    deg = dd[0, :_N, 0] + dd[1, :_N, 0] + 1.0
    denom = dd[0, :_N, 1] + dd[1, :_N, 1]
    dinv = jax.lax.rsqrt(deg + 1e-16)

    agg_a = _feature_agg_pass(src, dst, g, xr)
    agg_n = _feature_agg_pass(src, dst, ew, xr, dinv=dinv)
    agg_w = _feature_agg_pass(src, dst, ew, xr)
    agg = (agg_a, agg_n, agg_w)

    cnt = deg - 1.0
    nodevec = jnp.stack([denom, dinv, cnt, cnt], axis=1)
    params = jnp.concatenate([w, jnp.reshape(1.0 + eps, (1,))])
    return _combine(x, agg, nodevec, params, Ws, biases, relu_in)


def kernel(x, edge_index0, edge_logits0, edge_index1, edge_logits1, weights,
           gcn_W, gcn_b, gat_W, gat_a_src, gat_a_dst, gat_b,
           gin_W, gin_b, gin_eps, sage_Ws, sage_Wn, sage_b, lin_W, lin_b):
    h = x
    for i, (ei, el) in enumerate(((edge_index0, edge_logits0),
                                  (edge_index1, edge_logits1))):
        Ws = (gat_W[i], gcn_W[i], gin_W[i], sage_Ws[i], sage_Wn[i], lin_W[i])
        biases = jnp.stack([gat_b[i], gcn_b[i], gin_b[i], sage_b[i],
                            lin_b[i]], axis=0)
        h = _step(h, ei[0], ei[1], el, weights[i], gin_eps[i], Ws, biases,
                  gat_W[i], gat_a_src[i], gat_a_dst[i], relu_in=(i == 1))
    return h
